# baseline (speedup 1.0000x reference)
import functools

import jax
import jax.numpy as jnp
from jax import lax
from jax.experimental import pallas as pl
from jax.experimental.pallas import tpu as pltpu
from jax.experimental.pallas import tpu_sc as plsc

B = 16384
NF = 32

NC = 2
NS = 16
NW = NC * NS
BPW = B // NW
CH = 128
NCH = BPW // CH

_mesh = plsc.VectorSubcoreMesh(core_axis_name="c", subcore_axis_name="s")


@functools.partial(
    pl.kernel,
    mesh=_mesh,
    compiler_params=pltpu.CompilerParams(use_tc_tiling_on_sc=False),
    out_type=(
        jax.ShapeDtypeStruct((B, NF), jnp.float32),
        jax.ShapeDtypeStruct((B, NF), jnp.float32),
    ),
    scratch_types=[
        pltpu.VMEM((NCH, CH), jnp.int32),
        pltpu.VMEM((NCH, CH), jnp.int32),
        pltpu.VMEM((BPW, NF), jnp.float32),
        pltpu.VMEM((BPW, NF), jnp.float32),
        pltpu.SemaphoreType.DMA,
        pltpu.SemaphoreType.DMA,
    ],
)
def _gather_rows(uidx_hbm, pidx_hbm, p_hbm, q_hbm, p_out, q_out,
                 uidx_v, pidx_v, p_rows, q_rows, psem, qsem):
    wid = lax.axis_index("s") * NC + lax.axis_index("c")
    pltpu.sync_copy(uidx_hbm.at[wid], uidx_v)
    pltpu.sync_copy(pidx_hbm.at[wid], pidx_v)
    cps = []
    for j in range(NCH):
        cps.append(pltpu.async_copy(
            p_hbm.at[uidx_v.at[j]], p_rows.at[pl.ds(j * CH, CH)], psem))
        cps.append(pltpu.async_copy(
            q_hbm.at[pidx_v.at[j]], q_rows.at[pl.ds(j * CH, CH)], qsem))
    for cp in cps:
        cp.wait()
    base = wid * BPW
    pltpu.sync_copy(p_rows, p_out.at[pl.ds(base, BPW)])
    pltpu.sync_copy(q_rows, q_out.at[pl.ds(base, BPW)])


NB = 8
BM = B // NB


def _mlp_body(p_ref, q_ref, w1a_ref, w1b_ref, b1_ref, w2t_ref, b2_ref, out_ref):
    h = jnp.dot(p_ref[...], w1a_ref[...], preferred_element_type=jnp.float32)
    h = h + jnp.dot(q_ref[...], w1b_ref[...], preferred_element_type=jnp.float32)
    h = jnp.maximum(h + b1_ref[...], 0.0)
    out_ref[...] = jnp.sum(h * w2t_ref[...], axis=1, keepdims=True) + b2_ref[...]


_mlp = pl.pallas_call(
    _mlp_body,
    grid=(NB,),
    in_specs=[
        pl.BlockSpec((BM, NF), lambda i: (i, 0)),
        pl.BlockSpec((BM, NF), lambda i: (i, 0)),
        pl.BlockSpec((NF, NF), lambda i: (0, 0)),
        pl.BlockSpec((NF, NF), lambda i: (0, 0)),
        pl.BlockSpec((1, NF), lambda i: (0, 0)),
        pl.BlockSpec((1, NF), lambda i: (0, 0)),
        pl.BlockSpec((1, 1), lambda i: (0, 0)),
    ],
    out_specs=pl.BlockSpec((BM, 1), lambda i: (i, 0)),
    out_shape=jax.ShapeDtypeStruct((B, 1), jnp.float32),
)


def kernel(user, product, P_table, Q_table, W1, b1, W2, b2):
    uidx = user.reshape(NW, NCH, CH).astype(jnp.int32)
    pidx = product.reshape(NW, NCH, CH).astype(jnp.int32)
    p, q = _gather_rows(uidx, pidx, P_table, Q_table)
    return _mlp(p, q, W1[:NF], W1[NF:], b1.reshape(1, NF),
                W2.reshape(1, NF), b2.reshape(1, 1))
